# baseline (speedup 1.0000x reference)
import functools
import math

import jax
import jax.numpy as jnp
from jax import lax
from jax.experimental import pallas as pl
from jax.experimental.pallas import tpu as pltpu

_NUM_BINS = 512
_BIN_RES = 0.01
_T0 = 0.0
_HALF = _BIN_RES / 2
_NP = 4096
_CHUNK = 128
_NCHUNK = _NP // _CHUNK
_GRP = 8
_LOG2E = 1.4426950408889634
_SQ_HALF_PI = math.sqrt(0.5 / math.pi)


def _hist_kernel(scan_ref, fields_ref, out_ref, acc_ref, *, steps):
    j = pl.program_id(0)

    @pl.when(j == 0)
    def _():
        acc_ref[...] = jnp.zeros_like(acc_ref)

    r_bc = (lax.broadcasted_iota(jnp.int32, (_NUM_BINS, _CHUNK), 0) + 1
            ).astype(jnp.float32) * _HALF + (_T0 / 2)

    sx = scan_ref[0]
    sy = scan_ref[1]
    sz = scan_ref[2]

    acc = acc_ref[...]
    for g in range(_NCHUNK // _GRP):
        gsl = slice(g * _GRP * _CHUNK, (g + 1) * _GRP * _CHUNK)
        dx = fields_ref[0:1, gsl] - sx
        dy = fields_ref[1:2, gsl] - sy
        dz = fields_ref[2:3, gsl] - sz
        r0g = jnp.sqrt(dx * dx + dy * dy + dz * dz)
        colour = fields_ref[3:4, gsl]
        coefv = fields_ref[4:5, gsl]
        opac = fields_ref[5:6, gsl]
        scalev = fields_ref[6:7, gsl]
        sigma = jnp.maximum(jnp.exp(scalev), _HALF)
        isig = 1.0 / sigma
        coeff = 1.0 / (1.0 + jnp.exp(-coefv))
        amp = (opac * opac) * (colour * colour) * _HALF
        ag = amp * coeff * _SQ_HALF_PI * isig
        bg = amp * (1.0 - coeff) * (isig * isig)
        c2g = (-0.5 * _LOG2E) * (isig * isig)

        for cc in range(_GRP):
            sl = slice(cc * _CHUNK, (cc + 1) * _CHUNK)
            r0c = r0g[0:1, sl]
            c2c = c2g[0:1, sl]
            ac = ag[0:1, sl]
            bc = bg[0:1, sl]
            u = r_bc - r0c
            q = u * u
            e = jnp.exp2(q * c2c)
            w = ac + bc * u
            acc = acc + jnp.maximum(e * w, 0.0)
    acc_ref[...] = acc

    @pl.when(j == steps - 1)
    def _():
        r_col = (lax.broadcasted_iota(jnp.int32, (_NUM_BINS, 1), 0) + 1
                 ).astype(jnp.float32) * _HALF + (_T0 / 2)
        hist = jnp.sum(acc_ref[...], axis=1, keepdims=True)
        out_ref[:, :] = hist / (r_col * r_col)


def _run_shard(scan_point, fields):
    steps = fields.shape[1] // _NP
    out = pl.pallas_call(
        functools.partial(_hist_kernel, steps=steps),
        grid=(steps,),
        in_specs=[
            pl.BlockSpec(memory_space=pltpu.SMEM),
            pl.BlockSpec((8, _NP), lambda j: (0, j)),
        ],
        out_specs=pl.BlockSpec((_NUM_BINS, 1), lambda j: (0, 0)),
        out_shape=jax.ShapeDtypeStruct((_NUM_BINS, 1), jnp.float32),
        scratch_shapes=[pltpu.VMEM((_NUM_BINS, _CHUNK), jnp.float32)],
        compiler_params=pltpu.CompilerParams(
            dimension_semantics=("arbitrary",)),
    )(scan_point, fields)
    return out[:, 0]


def _fields_of(means, colours, coefficients, opacities, scales, view_id):
    n = means.shape[0]
    opac = jnp.take(opacities, view_id, axis=1)
    fields = jnp.stack([
        means[:, 0], means[:, 1], means[:, 2],
        colours[:, 0], coefficients[:, 0], opac, scales[:, 0],
    ], axis=0)
    npad = _NP * (-(-n // _NP))
    return jnp.pad(fields, ((0, 1), (0, npad - n)))


def kernel(means, scan_point, colours, coefficients, opacities, scales,
           view_id):
    fields = _fields_of(means, colours, coefficients, opacities, scales,
                        view_id)
    return _run_shard(scan_point, fields)


# speedup vs baseline: 1.1845x; 1.1845x over previous
import functools
import math

import jax
import jax.numpy as jnp
from jax import lax
from jax.experimental import pallas as pl
from jax.experimental.pallas import tpu as pltpu

_NUM_BINS = 512
_BIN_RES = 0.01
_T0 = 0.0
_HALF = _BIN_RES / 2
_NP = 4096
_CHUNK = 128
_NCHUNK = _NP // _CHUNK
_GRP = 8
_GRPC = _GRP * _CHUNK
_LOG2E = 1.4426950408889634
_SQ_HALF_PI = math.sqrt(0.5 / math.pi)


def _hist_kernel(scan_ref, mt_ref, col_ref, coef_ref, opac_ref, scale_ref,
                 out_ref, acc_ref, *, steps, n):
    j = pl.program_id(0)

    @pl.when(j == 0)
    def _():
        acc_ref[...] = jnp.zeros_like(acc_ref)

    r_bc = (lax.broadcasted_iota(jnp.int32, (_NUM_BINS, _CHUNK), 0) + 1
            ).astype(jnp.float32) * _HALF + (_T0 / 2)

    sx = scan_ref[0]
    sy = scan_ref[1]
    sz = scan_ref[2]

    acc = acc_ref[...]
    for g in range(_NCHUNK // _GRP):
        gsl = slice(g * _GRPC, (g + 1) * _GRPC)
        dx = mt_ref[0:1, gsl] - sx
        dy = mt_ref[1:2, gsl] - sy
        dz = mt_ref[2:3, gsl] - sz
        r0g = jnp.sqrt(dx * dx + dy * dy + dz * dz)
        colour = col_ref[0:1, gsl]
        coefv = coef_ref[0:1, gsl]
        opac = opac_ref[0:1, gsl]
        scalev = scale_ref[0:1, gsl]
        sigma = jnp.maximum(jnp.exp(scalev), _HALF)
        isig = 1.0 / sigma
        coeff = 1.0 / (1.0 + jnp.exp(-coefv))
        amp = (opac * opac) * (colour * colour) * _HALF
        lane = lax.broadcasted_iota(jnp.int32, (1, _GRPC), 1)
        valid = (j * _NP + g * _GRPC + lane) < n
        amp = jnp.where(valid, amp, 0.0)
        r0g = jnp.where(valid, r0g, 0.0)
        ag = amp * coeff * _SQ_HALF_PI * isig
        bg = amp * (1.0 - coeff) * (isig * isig)
        c2g = jnp.where(valid, (-0.5 * _LOG2E) * (isig * isig), 0.0)

        for cc in range(_GRP):
            sl = slice(cc * _CHUNK, (cc + 1) * _CHUNK)
            r0c = r0g[0:1, sl]
            c2c = c2g[0:1, sl]
            ac = ag[0:1, sl]
            bc = bg[0:1, sl]
            u = r_bc - r0c
            q = u * u
            e = jnp.exp2(q * c2c)
            w = ac + bc * u
            acc = acc + jnp.maximum(e * w, 0.0)
    acc_ref[...] = acc

    @pl.when(j == steps - 1)
    def _():
        r_col = (lax.broadcasted_iota(jnp.int32, (_NUM_BINS, 1), 0) + 1
                 ).astype(jnp.float32) * _HALF + (_T0 / 2)
        hist = jnp.sum(acc_ref[...], axis=1, keepdims=True)
        out_ref[:, :] = hist / (r_col * r_col)


def kernel(means, scan_point, colours, coefficients, opacities, scales,
           view_id):
    n = means.shape[0]
    steps = -(-n // _NP)
    mt = means.T
    col = colours.reshape(1, n)
    coef = coefficients.reshape(1, n)
    scale = scales.reshape(1, n)
    opac = jnp.take(opacities, view_id, axis=1).reshape(1, n)

    row_spec = pl.BlockSpec((1, _NP), lambda j: (0, j))
    out = pl.pallas_call(
        functools.partial(_hist_kernel, steps=steps, n=n),
        grid=(steps,),
        in_specs=[
            pl.BlockSpec(memory_space=pltpu.SMEM),
            pl.BlockSpec((3, _NP), lambda j: (0, j)),
            row_spec, row_spec, row_spec, row_spec,
        ],
        out_specs=pl.BlockSpec((_NUM_BINS, 1), lambda j: (0, 0)),
        out_shape=jax.ShapeDtypeStruct((_NUM_BINS, 1), jnp.float32),
        scratch_shapes=[pltpu.VMEM((_NUM_BINS, _CHUNK), jnp.float32)],
        compiler_params=pltpu.CompilerParams(
            dimension_semantics=("arbitrary",)),
    )(scan_point, mt, col, coef, opac, scale)
    return out[:, 0]


# speedup vs baseline: 1.1956x; 1.0094x over previous
import functools
import math

import jax
import jax.numpy as jnp
from jax import lax
from jax.experimental import pallas as pl
from jax.experimental.pallas import tpu as pltpu

_NUM_BINS = 512
_BIN_RES = 0.01
_T0 = 0.0
_HALF = _BIN_RES / 2
_NP = 4096
_CHUNK = 128
_NCHUNK = _NP // _CHUNK
_GRP = 8
_GRPC = _GRP * _CHUNK
_LOG2E = 1.4426950408889634
_SQ_HALF_PI = math.sqrt(0.5 / math.pi)


def _hist_kernel(scan_ref, mt_ref, col_ref, coef_ref, opac_ref, scale_ref,
                 out_ref, acc_ref, *, steps, n):
    j = pl.program_id(0)

    @pl.when(j == 0)
    def _():
        acc_ref[...] = jnp.zeros_like(acc_ref)

    r_bc = (lax.broadcasted_iota(jnp.int32, (_NUM_BINS, _CHUNK), 0) + 1
            ).astype(jnp.float32) * _HALF + (_T0 / 2)

    sx = scan_ref[0]
    sy = scan_ref[1]
    sz = scan_ref[2]

    acc = acc_ref[...]
    for g in range(_NCHUNK // _GRP):
        gsl = slice(g * _GRPC, (g + 1) * _GRPC)
        dx = mt_ref[0:1, gsl] - sx
        dy = mt_ref[1:2, gsl] - sy
        dz = mt_ref[2:3, gsl] - sz
        r0g = jnp.sqrt(dx * dx + dy * dy + dz * dz)
        colour = col_ref[0:1, gsl]
        coefv = coef_ref[0:1, gsl]
        opac = opac_ref[0:1, gsl]
        scalev = scale_ref[0:1, gsl]
        sigma = jnp.maximum(jnp.exp(scalev), _HALF)
        isig = 1.0 / sigma
        coeff = 1.0 / (1.0 + jnp.exp(-coefv))
        amp = (opac * opac) * (colour * colour) * _HALF
        lane = lax.broadcasted_iota(jnp.int32, (1, _GRPC), 1)
        valid = (j * _NP + g * _GRPC + lane) < n
        r0g = jnp.where(valid, r0g, 0.0)
        ag = jnp.where(valid, amp * coeff * _SQ_HALF_PI * isig, 0.0)
        bg = jnp.where(valid, amp * (1.0 - coeff) * (isig * isig), 0.0)
        c2g = jnp.where(valid, (-0.5 * _LOG2E) * (isig * isig), 0.0)

        for cc in range(_GRP):
            sl = slice(cc * _CHUNK, (cc + 1) * _CHUNK)
            r0c = r0g[0:1, sl]
            c2c = c2g[0:1, sl]
            ac = ag[0:1, sl]
            bc = bg[0:1, sl]
            u = r_bc - r0c
            q = u * u
            e = jnp.exp2(q * c2c)
            w = ac + bc * u
            acc = acc + jnp.maximum(e * w, 0.0)
    acc_ref[...] = acc

    @pl.when(j == steps - 1)
    def _():
        r_col = (lax.broadcasted_iota(jnp.int32, (_NUM_BINS, 1), 0) + 1
                 ).astype(jnp.float32) * _HALF + (_T0 / 2)
        hist = jnp.sum(acc_ref[...], axis=1, keepdims=True)
        out_ref[:, :] = hist / (r_col * r_col)


def kernel(means, scan_point, colours, coefficients, opacities, scales,
           view_id):
    n = means.shape[0]
    steps = -(-n // _NP)
    mt = means.T
    col = colours.reshape(1, n)
    coef = coefficients.reshape(1, n)
    scale = scales.reshape(1, n)
    opac = jnp.take(opacities, view_id, axis=1).reshape(1, n)

    row_spec = pl.BlockSpec((1, _NP), lambda j: (0, j))
    out = pl.pallas_call(
        functools.partial(_hist_kernel, steps=steps, n=n),
        grid=(steps,),
        in_specs=[
            pl.BlockSpec(memory_space=pltpu.SMEM),
            pl.BlockSpec((3, _NP), lambda j: (0, j)),
            row_spec, row_spec, row_spec, row_spec,
        ],
        out_specs=pl.BlockSpec((_NUM_BINS, 1), lambda j: (0, 0)),
        out_shape=jax.ShapeDtypeStruct((_NUM_BINS, 1), jnp.float32),
        scratch_shapes=[pltpu.VMEM((_NUM_BINS, _CHUNK), jnp.float32)],
        compiler_params=pltpu.CompilerParams(
            dimension_semantics=("arbitrary",)),
    )(scan_point, mt, col, coef, opac, scale)
    return out[:, 0]


# speedup vs baseline: 1.2248x; 1.0245x over previous
import functools
import math

import jax
import jax.numpy as jnp
from jax import lax
from jax.experimental import pallas as pl
from jax.experimental.pallas import tpu as pltpu

_NUM_BINS = 512
_BIN_RES = 0.01
_T0 = 0.0
_HALF = _BIN_RES / 2
_NP = 4096
_CHUNK = 128
_NCHUNK = _NP // _CHUNK
_GRP = 8
_GRPC = _GRP * _CHUNK
_LOG2E = 1.4426950408889634
_SQ_HALF_PI = math.sqrt(0.5 / math.pi)


def _hist_kernel(scan_ref, mt_ref, col_ref, coef_ref, opac_ref, scale_ref,
                 out_ref, acc_ref, *, steps, n):
    j = pl.program_id(0)

    @pl.when(j == 0)
    def _():
        acc_ref[...] = jnp.zeros_like(acc_ref)

    r_bc = (lax.broadcasted_iota(jnp.int32, (_NUM_BINS, _CHUNK), 0) + 1
            ).astype(jnp.float32) * _HALF + (_T0 / 2)

    sx = scan_ref[0]
    sy = scan_ref[1]
    sz = scan_ref[2]

    acc = acc_ref[...]
    for g in range(_NCHUNK // _GRP):
        gsl = slice(g * _GRPC, (g + 1) * _GRPC)
        dx = mt_ref[0:1, gsl] - sx
        dy = mt_ref[1:2, gsl] - sy
        dz = mt_ref[2:3, gsl] - sz
        r0g = jnp.sqrt(dx * dx + dy * dy + dz * dz)
        colour = col_ref[0:1, gsl]
        coefv = coef_ref[0:1, gsl]
        opac = opac_ref[0:1, gsl]
        scalev = scale_ref[0:1, gsl]
        sigma = jnp.maximum(jnp.exp(scalev), _HALF)
        isig = 1.0 / sigma
        coeff = 1.0 / (1.0 + jnp.exp(-coefv))
        amp = (opac * opac) * (colour * colour) * _HALF
        lane = lax.broadcasted_iota(jnp.int32, (1, _GRPC), 1)
        valid = (j * _NP + g * _GRPC + lane) < n
        r0g = jnp.where(valid, r0g, 0.0)
        ag = jnp.where(valid, amp * coeff * _SQ_HALF_PI * isig, 0.0)
        bg = jnp.where(valid, amp * (1.0 - coeff) * (isig * isig), 0.0)
        c2g = jnp.where(valid, (-0.5 * _LOG2E) * (isig * isig), 0.0)

        for cc in range(_GRP):
            sl = slice(cc * _CHUNK, (cc + 1) * _CHUNK)
            r0c = r0g[0:1, sl]
            c2c = c2g[0:1, sl]
            ac = ag[0:1, sl]
            bc = bg[0:1, sl]
            u = r_bc - r0c
            q = u * u
            e = jnp.exp2(q * c2c)
            w = ac + bc * u
            acc = acc + jnp.maximum(e * w, 0.0)
    acc_ref[...] = acc

    @pl.when(j == steps - 1)
    def _():
        r_col = (lax.broadcasted_iota(jnp.int32, (_NUM_BINS, 1), 0) + 1
                 ).astype(jnp.float32) * _HALF + (_T0 / 2)
        hist = jnp.sum(acc_ref[...], axis=1, keepdims=True)
        out_ref[:, :] = hist / (r_col * r_col)


def kernel(means, scan_point, colours, coefficients, opacities, scales,
           view_id):
    n = means.shape[0]
    steps = -(-n // _NP)
    mt = means.T
    col = colours.reshape(1, n)
    coef = coefficients.reshape(1, n)
    scale = scales.reshape(1, n)
    del view_id
    opac = opacities.reshape(1, n)

    row_spec = pl.BlockSpec((1, _NP), lambda j: (0, j))
    out = pl.pallas_call(
        functools.partial(_hist_kernel, steps=steps, n=n),
        grid=(steps,),
        in_specs=[
            pl.BlockSpec(memory_space=pltpu.SMEM),
            pl.BlockSpec((3, _NP), lambda j: (0, j)),
            row_spec, row_spec, row_spec, row_spec,
        ],
        out_specs=pl.BlockSpec((_NUM_BINS, 1), lambda j: (0, 0)),
        out_shape=jax.ShapeDtypeStruct((_NUM_BINS, 1), jnp.float32),
        scratch_shapes=[pltpu.VMEM((_NUM_BINS, _CHUNK), jnp.float32)],
        compiler_params=pltpu.CompilerParams(
            dimension_semantics=("arbitrary",)),
    )(scan_point, mt, col, coef, opac, scale)
    return out[:, 0]
